# baseline (speedup 1.0000x reference)
import jax
import jax.numpy as jnp
from jax.experimental import pallas as pl
from jax.experimental.pallas import tpu as pltpu

_B, _N, _F, _FE, _FG, _H, _FOUT, _STEPS = 8, 128, 128, 16, 128, 128, 128, 4
_TI = 32
_NT = _N // _TI


def _dot(a, b):
    return jax.lax.dot_general(a, b, (((1,), (0,)), ((), ())),
                               preferred_element_type=jnp.float32)


def _body(node_ref, edge_ref, graph_ref, adj_ref, wen_ref, wee_ref, weg_ref,
          m1_ref, m2_ref, me_ref, mg_ref, o1_ref, o2_ref, wd_ref,
          out_ref, me_s):
    bf16 = jnp.bfloat16
    node_h = _dot(node_ref[0], wen_ref[...])
    node_hb = node_h.astype(bf16)
    wfe = _dot(wee_ref[...], me_ref[...]).astype(bf16)
    wg = _dot(weg_ref[...], mg_ref[...]).astype(bf16)
    mg = _dot(graph_ref[0].astype(bf16), wg)

    edge = edge_ref[0]
    wfe_b = jnp.broadcast_to(wfe[None], (_TI, _FE, _H))
    for t in range(_NT):
        blk = edge[t * _TI:(t + 1) * _TI]
        me_s[t * _TI:(t + 1) * _TI] = jax.lax.dot_general(
            blk, wfe_b, (((1,), (1,)), ((0,), (0,))),
            preferred_element_type=jnp.float32).astype(bf16)

    m1w, m2w, o1w = m1_ref[...].astype(bf16), m2_ref[...].astype(bf16), o1_ref[...].astype(bf16)
    o2b, wdb = o2_ref[...].astype(bf16), wd_ref[...].astype(bf16)
    a1 = _dot(node_hb, m1w[:_H])
    a2 = _dot(node_hb, m2w[:_H]) + mg
    o1a = _dot(node_hb, o1w[:_H])
    adj = adj_ref[0]

    hid = None
    for _ in range(_STEPS):
        if hid is None:
            m1, m2, hl = a1, a2, o1a
        else:
            hb = hid.astype(bf16)
            m1 = a1 + _dot(hb, m1w[_H:])
            m2 = a2 + _dot(hb, m2w[_H:])
            hl = o1a + _dot(hb, o1w[_H:])
        m1b, m2b = m1.astype(bf16), m2.astype(bf16)
        aggs = []
        for t in range(_NT):
            sl = slice(t * _TI, (t + 1) * _TI)
            msgs = jnp.maximum(me_s[sl] + m1b[sl][:, None, :] + m2b[None, :, :],
                               bf16(0.0))
            aggs.append(jax.lax.dot_general(
                adj[sl], msgs, (((1,), (1,)), ((0,), (0,))),
                preferred_element_type=jnp.float32))
        agg = jnp.concatenate(aggs, axis=0)
        hid = jnp.maximum(hl + _dot(agg.astype(bf16), o2b), 0.0)

    out_ref[0] = _dot(node_hb, wdb[:_H]) + _dot(hid.astype(bf16), wdb[_H:])


def kernel(node_fts, edge_fts, graph_fts, adj, W_enc_node, W_enc_edge,
           W_enc_graph, M1, M2, Me, Mg, O1, O2, W_dec):
    bf16 = jnp.bfloat16
    graph3 = graph_fts.reshape(_B, 1, _FG)
    edge_t = edge_fts.astype(bf16).transpose(0, 1, 3, 2)
    adj_b = adj.astype(bf16)
    wspec = lambda *shape: pl.BlockSpec(shape, lambda b: (0,) * len(shape))
    return pl.pallas_call(
        _body,
        grid=(_B,),
        in_specs=[
            pl.BlockSpec((1, _N, _F), lambda b: (b, 0, 0)),
            pl.BlockSpec((1, _N, _FE, _N), lambda b: (b, 0, 0, 0)),
            pl.BlockSpec((1, 1, _FG), lambda b: (b, 0, 0)),
            pl.BlockSpec((1, _N, _N), lambda b: (b, 0, 0)),
            wspec(_F, _H),
            wspec(_FE, _H),
            wspec(_FG, _H),
            wspec(2 * _H, _H),
            wspec(2 * _H, _H),
            wspec(_H, _H),
            wspec(_H, _H),
            wspec(2 * _H, _H),
            wspec(_H, _H),
            wspec(2 * _H, _FOUT),
        ],
        out_specs=pl.BlockSpec((1, _N, _FOUT), lambda b: (b, 0, 0)),
        out_shape=jax.ShapeDtypeStruct((_B, _N, _FOUT), jnp.float32),
        scratch_shapes=[pltpu.VMEM((_N, _N, _H), jnp.bfloat16)],
        compiler_params=pltpu.CompilerParams(
            dimension_semantics=("arbitrary",)),
    )(node_fts, edge_t, graph3, adj_b, W_enc_node, W_enc_edge, W_enc_graph,
      M1, M2, Me, Mg, O1, O2, W_dec)


# speedup vs baseline: 1.0520x; 1.0520x over previous
import jax
import jax.numpy as jnp
from jax.experimental import pallas as pl
from jax.experimental.pallas import tpu as pltpu

_B, _N, _F, _FE, _FG, _H, _FOUT, _STEPS = 8, 128, 128, 16, 128, 128, 128, 4
_TI = 32
_NT = _N // _TI
_PB = 2


def _dot(a, b):
    return jax.lax.dot_general(a, b, (((1,), (0,)), ((), ())),
                               preferred_element_type=jnp.float32)


def _body(node_ref, edge_ref, graph_ref, adj_ref, wen_ref, wee_ref, weg_ref,
          m1_ref, m2_ref, me_ref, mg_ref, o1_ref, o2_ref, wd_ref,
          out_ref, *me_scratches):
    bf16 = jnp.bfloat16
    wfe = _dot(wee_ref[...], me_ref[...]).astype(bf16)
    wg = _dot(weg_ref[...], mg_ref[...])
    wfe_b = jnp.broadcast_to(wfe[None], (_TI, _FE, _H))
    m1w, m2w, o1w, wd = m1_ref[...], m2_ref[...], o1_ref[...], wd_ref[...]

    for i in range(_PB):
        me_s = me_scratches[i]
        node_h = _dot(node_ref[i], wen_ref[...])
        mg = _dot(graph_ref[i], wg)

        edge = edge_ref[i]
        for t in range(_NT):
            blk = edge[t * _TI:(t + 1) * _TI]
            me_s[t * _TI:(t + 1) * _TI] = jax.lax.dot_general(
                blk, wfe_b, (((1,), (1,)), ((0,), (0,))),
                preferred_element_type=jnp.float32).astype(bf16)

        a1 = _dot(node_h, m1w[:_H])
        a2 = _dot(node_h, m2w[:_H]) + mg
        o1a = _dot(node_h, o1w[:_H])
        adj = adj_ref[i].astype(bf16)

        hid = None
        for _ in range(_STEPS):
            if hid is None:
                m1, m2, hl = a1, a2, o1a
            else:
                m1 = a1 + _dot(hid, m1w[_H:])
                m2 = a2 + _dot(hid, m2w[_H:])
                hl = o1a + _dot(hid, o1w[_H:])
            m1b, m2b = m1.astype(bf16), m2.astype(bf16)
            aggs = []
            for t in range(_NT):
                sl = slice(t * _TI, (t + 1) * _TI)
                msgs = jnp.maximum(
                    me_s[sl] + m1b[sl][:, None, :] + m2b[None, :, :],
                    bf16(0.0))
                aggs.append(jax.lax.dot_general(
                    adj[sl], msgs, (((1,), (1,)), ((0,), (0,))),
                    preferred_element_type=jnp.float32))
            agg = jnp.concatenate(aggs, axis=0)
            hid = jnp.maximum(hl + _dot(agg, o2_ref[...]), 0.0)

        out_ref[i] = _dot(node_h, wd[:_H]) + _dot(hid, wd[_H:])


def kernel(node_fts, edge_fts, graph_fts, adj, W_enc_node, W_enc_edge,
           W_enc_graph, M1, M2, Me, Mg, O1, O2, W_dec):
    graph3 = graph_fts.reshape(_B, 1, _FG)
    edge_t = edge_fts.astype(jnp.bfloat16).transpose(0, 1, 3, 2)
    wspec = lambda *shape: pl.BlockSpec(shape, lambda b: (0,) * len(shape))
    return pl.pallas_call(
        _body,
        grid=(_B // _PB,),
        in_specs=[
            pl.BlockSpec((_PB, _N, _F), lambda b: (b, 0, 0)),
            pl.BlockSpec((_PB, _N, _FE, _N), lambda b: (b, 0, 0, 0)),
            pl.BlockSpec((_PB, 1, _FG), lambda b: (b, 0, 0)),
            pl.BlockSpec((_PB, _N, _N), lambda b: (b, 0, 0)),
            wspec(_F, _H),
            wspec(_FE, _H),
            wspec(_FG, _H),
            wspec(2 * _H, _H),
            wspec(2 * _H, _H),
            wspec(_H, _H),
            wspec(_H, _H),
            wspec(2 * _H, _H),
            wspec(_H, _H),
            wspec(2 * _H, _FOUT),
        ],
        out_specs=pl.BlockSpec((_PB, _N, _FOUT), lambda b: (b, 0, 0)),
        out_shape=jax.ShapeDtypeStruct((_B, _N, _FOUT), jnp.float32),
        scratch_shapes=[pltpu.VMEM((_N, _N, _H), jnp.bfloat16)
                        for _ in range(_PB)],
        compiler_params=pltpu.CompilerParams(
            dimension_semantics=("arbitrary",)),
    )(node_fts, edge_t, graph3, adj, W_enc_node, W_enc_edge, W_enc_graph,
      M1, M2, Me, Mg, O1, O2, W_dec)


# speedup vs baseline: 1.0686x; 1.0158x over previous
import jax
import jax.numpy as jnp
from jax.experimental import pallas as pl
from jax.experimental.pallas import tpu as pltpu

_B, _N, _F, _FE, _FG, _H, _FOUT, _STEPS = 8, 128, 128, 16, 128, 128, 128, 4
_TI = 32
_NT = _N // _TI
_PB = 2


def _dot(a, b):
    return jax.lax.dot_general(a, b, (((1,), (0,)), ((), ())),
                               preferred_element_type=jnp.float32)


def _body(node_ref, edge_ref, graph_ref, adj_ref, wen_ref, wee_ref, weg_ref,
          m1_ref, m2_ref, me_ref, mg_ref, o1_ref, o2_ref, wd_ref,
          out_ref, *me_scratches):
    bf16 = jnp.bfloat16
    wfe = _dot(wee_ref[...], me_ref[...]).astype(bf16)
    wg = _dot(weg_ref[...], mg_ref[...])
    wfe_b = jnp.broadcast_to(wfe[None], (_TI, _FE, _H))
    m1w, m2w, o1w, wd = m1_ref[...], m2_ref[...], o1_ref[...], wd_ref[...]

    for i in range(_PB):
        me_s = me_scratches[i]
        node_h = _dot(node_ref[i], wen_ref[...])
        mg = _dot(graph_ref[i], wg)

        edge = edge_ref[i]
        for t in range(_NT):
            blk = edge[t * _TI:(t + 1) * _TI]
            me_s[t * _TI:(t + 1) * _TI] = jax.lax.dot_general(
                blk, wfe_b, (((1,), (1,)), ((0,), (0,))),
                preferred_element_type=jnp.float32).astype(bf16)

        a1 = _dot(node_h, m1w[:_H])
        a2 = _dot(node_h, m2w[:_H]) + mg
        o1a = _dot(node_h, o1w[:_H])
        adj_f = adj_ref[i]
        adj = adj_f.astype(bf16)
        rs = jnp.sum(adj_f, axis=1, keepdims=True)

        hid = None
        for _ in range(_STEPS):
            if hid is None:
                m1, m2, hl = a1, a2, o1a
            else:
                m1 = a1 + _dot(hid, m1w[_H:])
                m2 = a2 + _dot(hid, m2w[_H:])
                hl = o1a + _dot(hid, o1w[_H:])
            nm1b, m2b = (-m1).astype(bf16), m2.astype(bf16)
            aggs = []
            for t in range(_NT):
                sl = slice(t * _TI, (t + 1) * _TI)
                msgs = jnp.maximum(
                    me_s[sl] + m2b[None, :, :],
                    nm1b[sl][:, None, :])
                aggs.append(jax.lax.dot_general(
                    adj[sl], msgs, (((1,), (1,)), ((0,), (0,))),
                    preferred_element_type=jnp.float32))
            agg = jnp.concatenate(aggs, axis=0) + m1 * rs
            hid = jnp.maximum(hl + _dot(agg, o2_ref[...]), 0.0)

        out_ref[i] = _dot(node_h, wd[:_H]) + _dot(hid, wd[_H:])


def kernel(node_fts, edge_fts, graph_fts, adj, W_enc_node, W_enc_edge,
           W_enc_graph, M1, M2, Me, Mg, O1, O2, W_dec):
    graph3 = graph_fts.reshape(_B, 1, _FG)
    edge_t = edge_fts.astype(jnp.bfloat16).transpose(0, 1, 3, 2)
    wspec = lambda *shape: pl.BlockSpec(shape, lambda b: (0,) * len(shape))
    return pl.pallas_call(
        _body,
        grid=(_B // _PB,),
        in_specs=[
            pl.BlockSpec((_PB, _N, _F), lambda b: (b, 0, 0)),
            pl.BlockSpec((_PB, _N, _FE, _N), lambda b: (b, 0, 0, 0)),
            pl.BlockSpec((_PB, 1, _FG), lambda b: (b, 0, 0)),
            pl.BlockSpec((_PB, _N, _N), lambda b: (b, 0, 0)),
            wspec(_F, _H),
            wspec(_FE, _H),
            wspec(_FG, _H),
            wspec(2 * _H, _H),
            wspec(2 * _H, _H),
            wspec(_H, _H),
            wspec(_H, _H),
            wspec(2 * _H, _H),
            wspec(_H, _H),
            wspec(2 * _H, _FOUT),
        ],
        out_specs=pl.BlockSpec((_PB, _N, _FOUT), lambda b: (b, 0, 0)),
        out_shape=jax.ShapeDtypeStruct((_B, _N, _FOUT), jnp.float32),
        scratch_shapes=[pltpu.VMEM((_N, _N, _H), jnp.bfloat16)
                        for _ in range(_PB)],
        compiler_params=pltpu.CompilerParams(
            dimension_semantics=("arbitrary",)),
    )(node_fts, edge_t, graph3, adj, W_enc_node, W_enc_edge, W_enc_graph,
      M1, M2, Me, Mg, O1, O2, W_dec)
